# Initial kernel scaffold; baseline (speedup 1.0000x reference)
import jax
import jax.numpy as jnp
from jax.experimental import pallas as pl


def kernel(x, edge_index, edge_time, edge_attr, params):
    raise NotImplementedError("write your pallas kernel here")


# speedup vs baseline: 2.7914x; 2.7914x over previous
import jax
import jax.numpy as jnp
from jax import lax
from jax.experimental import pallas as pl
from jax.experimental.pallas import tpu as pltpu
from jax.experimental.pallas import tpu_sc as plsc

N = 10000
E = 320000
D_IN = 128
D_OUT = 128
D_EDGE = 16
HEADS = 4
D_K = D_OUT // HEADS
TIME_UNIT = 86400.0
SCALE = D_K ** -0.5
SELU_ALPHA = 1.6732632423543772
SELU_SCALE = 1.0507009873554805

NC = 2
NS = 16
NW = NC * NS
CHUNK = 64
NCHUNK = 158
PER_W = NCHUNK * CHUNK
EPAD = NW * PER_W
BE = 2048
BN = 2000
NPAD = 10240
NPS = NPAD // NS
NZC = 64
ETB = NPAD // 32


def _gather_body(x_hbm, idx_hbm, out_hbm, idxv, rows, sem):
    c = lax.axis_index("c")
    s = lax.axis_index("s")
    wid = s * NC + c
    for t in range(2):
        def body(ci, _):
            pltpu.sync_copy(idx_hbm.at[t, wid, ci, 0], idxv)
            pltpu.async_copy(x_hbm.at[idxv], rows, sem).wait()
            pltpu.sync_copy(rows, out_hbm.at[t, pl.ds(wid * PER_W + ci * CHUNK, CHUNK)])
            return 0

        lax.fori_loop(0, NCHUNK, body, 0)


def _scatter_body(idx_hbm, idx2_hbm, zidx_hbm, zidx2_hbm, wv_hbm, ex_hbm,
                  zero_hbm, aggw_hbm, agge_hbm, idxv, idx2v, wvbuf, exbuf, shw, she):
    c = lax.axis_index("c")
    s = lax.axis_index("s")
    wid = s * NC + c
    pltpu.sync_copy(zero_hbm, wvbuf)
    for r in range(NPS // NZC):
        pltpu.sync_copy(zidx_hbm.at[s, r, 0], idxv)
        pltpu.sync_copy(wvbuf, shw.at[idxv])

    @pl.when(s == 0)
    def _zero_e():
        for r in range(ETB // NZC):
            pltpu.sync_copy(zidx2_hbm.at[r, 0], idxv)
            pltpu.sync_copy(wvbuf, she.at[idxv])

    plsc.subcore_barrier()

    def body(ci, _):
        base = wid * PER_W + ci * CHUNK
        pltpu.sync_copy(idx_hbm.at[wid, ci, 0], idxv)
        pltpu.sync_copy(idx2_hbm.at[wid, ci, 0], idx2v)
        pltpu.sync_copy(wv_hbm.at[pl.ds(base, CHUNK)], wvbuf)
        pltpu.sync_copy(ex_hbm.at[pl.ds(base, CHUNK)], exbuf)
        pltpu.sync_copy(wvbuf, shw.at[idxv], add=True)
        pltpu.sync_copy(exbuf, she.at[idx2v], add=True)
        return 0

    lax.fori_loop(0, NCHUNK, body, 0)
    plsc.subcore_barrier()
    for r in range(NPS // NZC):
        pltpu.sync_copy(zidx_hbm.at[s, r, 0], idxv)
        pltpu.sync_copy(shw.at[idxv], wvbuf)
        pltpu.sync_copy(wvbuf, aggw_hbm.at[c, pl.ds(s * NPS + r * NZC, NZC)])

    @pl.when(s == 0)
    def _dump_e():
        for r in range(ETB // NZC):
            pltpu.sync_copy(zidx2_hbm.at[r, 0], idxv)
            pltpu.sync_copy(she.at[idxv], exbuf)
            pltpu.sync_copy(exbuf, agge_hbm.at[c, pl.ds(r * NZC, NZC)])


def _edge_body(g_ref, ea_ref, dt_ref, ei_ref,
               wq_ref, bq_ref,
               ktop_ref, kbot_ref, kbias_ref, kc_ref, kla_ref,
               vtop_ref, vbot_ref, vbias_ref, vc_ref, vla_ref,
               sdk_ref, shd_ref, rmat_ref, tmat_ref,
               wv_ref, ex_ref):
    xi = g_ref[0]
    xj = g_ref[1]
    ea = ea_ref[...]
    dtn = dt_ref[...] / TIME_UNIT

    q = jnp.maximum(
        jnp.dot(xi, wq_ref[...], preferred_element_type=jnp.float32) + bq_ref[...], 0.0)

    def gate(top_ref, bot_ref, bias_ref, c_ref, la_ref):
        u = (jnp.dot(xj, top_ref[...], preferred_element_type=jnp.float32)
             + jnp.dot(ea, bot_ref[...], preferred_element_type=jnp.float32)
             + bias_ref[...])
        state = u[:, :D_OUT]
        bu = u[:, D_OUT:2 * D_OUT]
        outl = u[:, 2 * D_OUT:]
        a = -jnp.exp(la_ref[...])
        a_zoh = jnp.exp(dtn * a)
        pre = a_zoh * state + dtn * bu
        z = jnp.dot(pre, c_ref[...], preferred_element_type=jnp.float32)
        g = SELU_SCALE * jnp.where(
            z > 0.0, z, SELU_ALPHA * (jnp.exp(jnp.minimum(z, 0.0)) - 1.0))
        return g * outl

    k = jnp.maximum(gate(ktop_ref, kbot_ref, kbias_ref, kc_ref, kla_ref), 0.0)
    v = gate(vtop_ref, vbot_ref, vbias_ref, vc_ref, vla_ref)

    res = jnp.dot(q * k, sdk_ref[...], preferred_element_type=jnp.float32) * SCALE
    ex = jnp.exp(res)
    wv = jnp.dot(ex, shd_ref[...], preferred_element_type=jnp.float32) * v

    b = pl.program_id(0)
    rowid = b * BE + lax.broadcasted_iota(jnp.int32, (BE, 1), 0)
    m = jnp.where(rowid < E, 1.0, 0.0)
    imod = ei_ref[...] % 32
    onehot = jnp.where(
        imod == lax.broadcasted_iota(jnp.int32, (BE, 32), 1), 1.0, 0.0)
    ex128 = (jnp.dot(onehot, rmat_ref[...], preferred_element_type=jnp.float32)
             * jnp.dot(ex, tmat_ref[...], preferred_element_type=jnp.float32))
    wv_ref[...] = wv * m
    ex_ref[...] = ex128 * m


def _final_body(x_ref, ag_ref, de_ref, m1a_ref, m1b_ref, b1_ref, m2_ref, b2_ref,
                shd_ref, out_ref):
    xb = x_ref[...]
    agg = ag_ref[0] + ag_ref[1]
    den = de_ref[0] + de_ref[1]
    rec = 1.0 / (den + 1e-16)
    aggn = agg * jnp.dot(rec, shd_ref[...], preferred_element_type=jnp.float32)
    h = jnp.maximum(
        jnp.dot(xb, m1a_ref[...], preferred_element_type=jnp.float32)
        + jnp.dot(aggn, m1b_ref[...], preferred_element_type=jnp.float32)
        + b1_ref[...], 0.0)
    out_ref[...] = (jnp.dot(h, m2_ref[...], preferred_element_type=jnp.float32)
                    + b2_ref[...] + xb)


def _full(shape):
    nd = len(shape)
    return pl.BlockSpec(shape, lambda b: (0,) * nd)


def kernel(x, edge_index, edge_time, edge_attr, params):
    f32 = jnp.float32
    pad = EPAD - E
    ipad = jnp.concatenate([edge_index[0], jnp.zeros((pad,), jnp.int32)])
    jpad = jnp.concatenate([edge_index[1], jnp.zeros((pad,), jnp.int32)])
    idx_g = jnp.stack([ipad, jpad]).reshape(2, NW, NCHUNK, 1, CHUNK)
    idx_s = ipad.reshape(NW, NCHUNK, 1, CHUNK)
    idx2_s = (ipad // 32).reshape(NW, NCHUNK, 1, CHUNK)
    zidx = jnp.arange(NPAD, dtype=jnp.int32).reshape(NS, NPS // NZC, 1, NZC)
    zidx2 = jnp.arange(ETB, dtype=jnp.int32).reshape(ETB // NZC, 1, NZC)
    ei = ipad.reshape(EPAD, 1)
    ea = jnp.pad(edge_attr, ((0, pad), (0, 0)))
    dt = jnp.pad(edge_time, (0, pad)).reshape(EPAD, 1)
    zero_tab = jnp.zeros((NZC, D_OUT), f32)

    mesh = plsc.VectorSubcoreMesh(core_axis_name="c", subcore_axis_name="s")
    gath = pl.kernel(
        _gather_body,
        out_type=jax.ShapeDtypeStruct((2, EPAD, D_IN), f32),
        mesh=mesh,
        scratch_types=[
            pltpu.VMEM((CHUNK,), jnp.int32),
            pltpu.VMEM((CHUNK, D_IN), f32),
            pltpu.SemaphoreType.DMA,
        ],
    )(x, idx_g)

    def gate_mats(p):
        top = jnp.concatenate(
            [p["W_state"][:D_IN], p["B"][:D_IN], p["W_out"][:D_IN]], axis=1)
        bot = jnp.concatenate(
            [p["W_state"][D_IN:], p["B"][D_IN:], p["W_out"][D_IN:]], axis=1)
        bias = jnp.concatenate(
            [p["b_state"], jnp.zeros((D_OUT,), f32), p["b_out"]]).reshape(1, 3 * D_OUT)
        return top, bot, bias, p["C"], p["log_nA"].reshape(1, D_OUT)

    ktop, kbot, kbias, kc, kla = gate_mats(params["k"])
    vtop, vbot, vbias, vc, vla = gate_mats(params["v"])
    heads = jnp.arange(HEADS, dtype=jnp.int32)
    sdk = (jnp.arange(D_OUT, dtype=jnp.int32)[:, None] // D_K == heads[None, :]).astype(f32)
    shd = sdk.T
    lanes = jnp.arange(D_OUT, dtype=jnp.int32)
    rmat = (jnp.arange(32, dtype=jnp.int32)[:, None] == lanes[None, :] // 4).astype(f32)
    tmat = (heads[:, None] == lanes[None, :] % 4).astype(f32)
    bq = params["bq"].reshape(1, D_OUT)

    weights = [params["Wq"], bq, ktop, kbot, kbias, kc, kla,
               vtop, vbot, vbias, vc, vla, sdk, shd, rmat, tmat]

    wv, ex128 = pl.pallas_call(
        _edge_body,
        grid=(EPAD // BE,),
        in_specs=[
            pl.BlockSpec((2, BE, D_IN), lambda b: (0, b, 0)),
            pl.BlockSpec((BE, D_EDGE), lambda b: (b, 0)),
            pl.BlockSpec((BE, 1), lambda b: (b, 0)),
            pl.BlockSpec((BE, 1), lambda b: (b, 0)),
        ] + [_full(w.shape) for w in weights],
        out_specs=[
            pl.BlockSpec((BE, D_OUT), lambda b: (b, 0)),
            pl.BlockSpec((BE, D_OUT), lambda b: (b, 0)),
        ],
        out_shape=[
            jax.ShapeDtypeStruct((EPAD, D_OUT), f32),
            jax.ShapeDtypeStruct((EPAD, D_OUT), f32),
        ],
    )(gath, ea, dt, ei, *weights)

    aggw, agge = pl.kernel(
        _scatter_body,
        out_type=[
            jax.ShapeDtypeStruct((NC, NPAD, D_OUT), f32),
            jax.ShapeDtypeStruct((NC, ETB, D_OUT), f32),
        ],
        mesh=mesh,
        scratch_types=[
            pltpu.VMEM((CHUNK,), jnp.int32),
            pltpu.VMEM((CHUNK,), jnp.int32),
            pltpu.VMEM((CHUNK, D_OUT), f32),
            pltpu.VMEM((CHUNK, D_OUT), f32),
            pltpu.VMEM_SHARED((NPAD, D_OUT), f32),
            pltpu.VMEM_SHARED((ETB, D_OUT), f32),
        ],
    )(idx_s, idx2_s, zidx, zidx2, wv, ex128, zero_tab)
    den_tab = agge.reshape(NC, NPAD, HEADS)

    m1a = params["m1_W"][:D_IN]
    m1b = params["m1_W"][D_IN:]
    b1 = params["m1_b"].reshape(1, D_IN)
    b2 = params["m2_b"].reshape(1, D_OUT)
    fw = [m1a, m1b, b1, params["m2_W"], b2, shd]

    out = pl.pallas_call(
        _final_body,
        grid=(N // BN,),
        in_specs=[
            pl.BlockSpec((BN, D_IN), lambda b: (b, 0)),
            pl.BlockSpec((NC, BN, D_OUT), lambda b: (0, b, 0)),
            pl.BlockSpec((NC, BN, HEADS), lambda b: (0, b, 0)),
        ] + [_full(w.shape) for w in fw],
        out_specs=pl.BlockSpec((BN, D_OUT), lambda b: (b, 0)),
        out_shape=jax.ShapeDtypeStruct((N, D_OUT), f32),
    )(x, aggw, den_tab, *fw)
    return out


# speedup vs baseline: 2.9496x; 1.0567x over previous
import jax
import jax.numpy as jnp
from jax import lax
from jax.experimental import pallas as pl
from jax.experimental.pallas import tpu as pltpu
from jax.experimental.pallas import tpu_sc as plsc

N = 10000
E = 320000
D_IN = 128
D_OUT = 128
D_EDGE = 16
HEADS = 4
D_K = D_OUT // HEADS
TIME_UNIT = 86400.0
SCALE = D_K ** -0.5
SELU_ALPHA = 1.6732632423543772
SELU_SCALE = 1.0507009873554805

NC = 2
NS = 16
NW = NC * NS
CHUNK = 64
NCHUNK = 158
PER_W = NCHUNK * CHUNK
EPAD = NW * PER_W
BE = 2048
BN = 2000
NPAD = 10240
NPS = NPAD // NS
NZC = 64
ETB = NPAD // 32
G_CHUNK = 128
G_NCHUNK = PER_W // G_CHUNK


def _gather_body(x_hbm, idx_hbm, out_hbm, idxv, rows, sem):
    c = lax.axis_index("c")
    s = lax.axis_index("s")
    wid = s * NC + c
    for t in range(2):
        def body(ci, _):
            pltpu.sync_copy(idx_hbm.at[t, wid, ci, 0], idxv)
            pltpu.async_copy(x_hbm.at[idxv], rows, sem).wait()
            pltpu.sync_copy(
                rows, out_hbm.at[t, pl.ds(wid * PER_W + ci * G_CHUNK, G_CHUNK)])
            return 0

        lax.fori_loop(0, G_NCHUNK, body, 0)


def _scatter_body(idx_hbm, idx2_hbm, zidx_hbm, zidx2_hbm, wv_hbm, ex_hbm,
                  zero_hbm, aggw_hbm, agge_hbm, idxv, idx2v, wvbuf, exbuf, shw, she):
    c = lax.axis_index("c")
    s = lax.axis_index("s")
    wid = s * NC + c
    pltpu.sync_copy(zero_hbm, wvbuf)
    for r in range(NPS // NZC):
        pltpu.sync_copy(zidx_hbm.at[s, r, 0], idxv)
        pltpu.sync_copy(wvbuf, shw.at[idxv])

    @pl.when(s == 0)
    def _zero_e():
        for r in range(ETB // NZC):
            pltpu.sync_copy(zidx2_hbm.at[r, 0], idxv)
            pltpu.sync_copy(wvbuf, she.at[idxv])

    plsc.subcore_barrier()

    def body(ci, _):
        base = wid * PER_W + ci * CHUNK
        pltpu.sync_copy(idx_hbm.at[wid, ci, 0], idxv)
        pltpu.sync_copy(idx2_hbm.at[wid, ci, 0], idx2v)
        pltpu.sync_copy(wv_hbm.at[pl.ds(base, CHUNK)], wvbuf)
        pltpu.sync_copy(ex_hbm.at[pl.ds(base, CHUNK)], exbuf)
        pltpu.sync_copy(wvbuf, shw.at[idxv], add=True)
        pltpu.sync_copy(exbuf, she.at[idx2v], add=True)
        return 0

    lax.fori_loop(0, NCHUNK, body, 0)
    plsc.subcore_barrier()
    for r in range(NPS // NZC):
        pltpu.sync_copy(zidx_hbm.at[s, r, 0], idxv)
        pltpu.sync_copy(shw.at[idxv], wvbuf)
        pltpu.sync_copy(wvbuf, aggw_hbm.at[c, pl.ds(s * NPS + r * NZC, NZC)])

    @pl.when(s == 0)
    def _dump_e():
        for r in range(ETB // NZC):
            pltpu.sync_copy(zidx2_hbm.at[r, 0], idxv)
            pltpu.sync_copy(she.at[idxv], exbuf)
            pltpu.sync_copy(exbuf, agge_hbm.at[c, pl.ds(r * NZC, NZC)])


def _edge_body(g_ref, ea_ref, dt_ref, ei_ref,
               wq_ref, bq_ref,
               ktop_ref, kbot_ref, kbias_ref, kc_ref, kla_ref,
               vtop_ref, vbot_ref, vbias_ref, vc_ref, vla_ref,
               sdk_ref, shd_ref, rmat_ref, tmat_ref,
               wv_ref, ex_ref):
    xi = g_ref[0]
    xj = g_ref[1]
    ea = ea_ref[...]
    dtn = dt_ref[...] / TIME_UNIT

    q = jnp.maximum(
        jnp.dot(xi, wq_ref[...], preferred_element_type=jnp.float32) + bq_ref[...], 0.0)

    def gate(top_ref, bot_ref, bias_ref, c_ref, la_ref):
        u = (jnp.dot(xj, top_ref[...], preferred_element_type=jnp.float32)
             + jnp.dot(ea, bot_ref[...], preferred_element_type=jnp.float32)
             + bias_ref[...])
        state = u[:, :D_OUT]
        bu = u[:, D_OUT:2 * D_OUT]
        outl = u[:, 2 * D_OUT:]
        a = -jnp.exp(la_ref[...])
        a_zoh = jnp.exp(dtn * a)
        pre = a_zoh * state + dtn * bu
        z = jnp.dot(pre, c_ref[...], preferred_element_type=jnp.float32)
        g = SELU_SCALE * jnp.where(
            z > 0.0, z, SELU_ALPHA * (jnp.exp(jnp.minimum(z, 0.0)) - 1.0))
        return g * outl

    k = jnp.maximum(gate(ktop_ref, kbot_ref, kbias_ref, kc_ref, kla_ref), 0.0)
    v = gate(vtop_ref, vbot_ref, vbias_ref, vc_ref, vla_ref)

    res = jnp.dot(q * k, sdk_ref[...], preferred_element_type=jnp.float32) * SCALE
    ex = jnp.exp(res)
    wv = jnp.dot(ex, shd_ref[...], preferred_element_type=jnp.float32) * v

    b = pl.program_id(0)
    rowid = b * BE + lax.broadcasted_iota(jnp.int32, (BE, 1), 0)
    m = jnp.where(rowid < E, 1.0, 0.0)
    imod = ei_ref[...] % 32
    onehot = jnp.where(
        imod == lax.broadcasted_iota(jnp.int32, (BE, 32), 1), 1.0, 0.0)
    ex128 = (jnp.dot(onehot, rmat_ref[...], preferred_element_type=jnp.float32)
             * jnp.dot(ex, tmat_ref[...], preferred_element_type=jnp.float32))
    wv_ref[...] = wv * m
    ex_ref[...] = ex128 * m


def _final_body(x_ref, ag_ref, de_ref, m1a_ref, m1b_ref, b1_ref, m2_ref, b2_ref,
                shd_ref, out_ref):
    xb = x_ref[...]
    agg = ag_ref[0] + ag_ref[1]
    den = de_ref[0] + de_ref[1]
    rec = 1.0 / (den + 1e-16)
    aggn = agg * jnp.dot(rec, shd_ref[...], preferred_element_type=jnp.float32)
    h = jnp.maximum(
        jnp.dot(xb, m1a_ref[...], preferred_element_type=jnp.float32)
        + jnp.dot(aggn, m1b_ref[...], preferred_element_type=jnp.float32)
        + b1_ref[...], 0.0)
    out_ref[...] = (jnp.dot(h, m2_ref[...], preferred_element_type=jnp.float32)
                    + b2_ref[...] + xb)


def _full(shape):
    nd = len(shape)
    return pl.BlockSpec(shape, lambda b: (0,) * nd)


def kernel(x, edge_index, edge_time, edge_attr, params):
    f32 = jnp.float32
    pad = EPAD - E
    ipad = jnp.concatenate([edge_index[0], jnp.zeros((pad,), jnp.int32)])
    jpad = jnp.concatenate([edge_index[1], jnp.zeros((pad,), jnp.int32)])
    idx_g = jnp.stack([ipad, jpad]).reshape(2, NW, G_NCHUNK, 1, G_CHUNK)
    idx_s = ipad.reshape(NW, NCHUNK, 1, CHUNK)
    idx2_s = (ipad // 32).reshape(NW, NCHUNK, 1, CHUNK)
    zidx = jnp.arange(NPAD, dtype=jnp.int32).reshape(NS, NPS // NZC, 1, NZC)
    zidx2 = jnp.arange(ETB, dtype=jnp.int32).reshape(ETB // NZC, 1, NZC)
    ei = ipad.reshape(EPAD, 1)
    ea = jnp.pad(edge_attr, ((0, pad), (0, 0)))
    dt = jnp.pad(edge_time, (0, pad)).reshape(EPAD, 1)
    zero_tab = jnp.zeros((NZC, D_OUT), f32)

    mesh = plsc.VectorSubcoreMesh(core_axis_name="c", subcore_axis_name="s")
    gath = pl.kernel(
        _gather_body,
        out_type=jax.ShapeDtypeStruct((2, EPAD, D_IN), f32),
        mesh=mesh,
        scratch_types=[
            pltpu.VMEM((G_CHUNK,), jnp.int32),
            pltpu.VMEM((G_CHUNK, D_IN), f32),
            pltpu.SemaphoreType.DMA,
        ],
    )(x, idx_g)

    def gate_mats(p):
        top = jnp.concatenate(
            [p["W_state"][:D_IN], p["B"][:D_IN], p["W_out"][:D_IN]], axis=1)
        bot = jnp.concatenate(
            [p["W_state"][D_IN:], p["B"][D_IN:], p["W_out"][D_IN:]], axis=1)
        bias = jnp.concatenate(
            [p["b_state"], jnp.zeros((D_OUT,), f32), p["b_out"]]).reshape(1, 3 * D_OUT)
        return top, bot, bias, p["C"], p["log_nA"].reshape(1, D_OUT)

    ktop, kbot, kbias, kc, kla = gate_mats(params["k"])
    vtop, vbot, vbias, vc, vla = gate_mats(params["v"])
    heads = jnp.arange(HEADS, dtype=jnp.int32)
    sdk = (jnp.arange(D_OUT, dtype=jnp.int32)[:, None] // D_K == heads[None, :]).astype(f32)
    shd = sdk.T
    lanes = jnp.arange(D_OUT, dtype=jnp.int32)
    rmat = (jnp.arange(32, dtype=jnp.int32)[:, None] == lanes[None, :] // 4).astype(f32)
    tmat = (heads[:, None] == lanes[None, :] % 4).astype(f32)
    bq = params["bq"].reshape(1, D_OUT)

    weights = [params["Wq"], bq, ktop, kbot, kbias, kc, kla,
               vtop, vbot, vbias, vc, vla, sdk, shd, rmat, tmat]

    wv, ex128 = pl.pallas_call(
        _edge_body,
        grid=(EPAD // BE,),
        in_specs=[
            pl.BlockSpec((2, BE, D_IN), lambda b: (0, b, 0)),
            pl.BlockSpec((BE, D_EDGE), lambda b: (b, 0)),
            pl.BlockSpec((BE, 1), lambda b: (b, 0)),
            pl.BlockSpec((BE, 1), lambda b: (b, 0)),
        ] + [_full(w.shape) for w in weights],
        out_specs=[
            pl.BlockSpec((BE, D_OUT), lambda b: (b, 0)),
            pl.BlockSpec((BE, D_OUT), lambda b: (b, 0)),
        ],
        out_shape=[
            jax.ShapeDtypeStruct((EPAD, D_OUT), f32),
            jax.ShapeDtypeStruct((EPAD, D_OUT), f32),
        ],
    )(gath, ea, dt, ei, *weights)

    aggw, agge = pl.kernel(
        _scatter_body,
        out_type=[
            jax.ShapeDtypeStruct((NC, NPAD, D_OUT), f32),
            jax.ShapeDtypeStruct((NC, ETB, D_OUT), f32),
        ],
        mesh=mesh,
        scratch_types=[
            pltpu.VMEM((CHUNK,), jnp.int32),
            pltpu.VMEM((CHUNK,), jnp.int32),
            pltpu.VMEM((CHUNK, D_OUT), f32),
            pltpu.VMEM((CHUNK, D_OUT), f32),
            pltpu.VMEM_SHARED((NPAD, D_OUT), f32),
            pltpu.VMEM_SHARED((ETB, D_OUT), f32),
        ],
    )(idx_s, idx2_s, zidx, zidx2, wv, ex128, zero_tab)
    den_tab = agge.reshape(NC, NPAD, HEADS)

    m1a = params["m1_W"][:D_IN]
    m1b = params["m1_W"][D_IN:]
    b1 = params["m1_b"].reshape(1, D_IN)
    b2 = params["m2_b"].reshape(1, D_OUT)
    fw = [m1a, m1b, b1, params["m2_W"], b2, shd]

    out = pl.pallas_call(
        _final_body,
        grid=(N // BN,),
        in_specs=[
            pl.BlockSpec((BN, D_IN), lambda b: (b, 0)),
            pl.BlockSpec((NC, BN, D_OUT), lambda b: (0, b, 0)),
            pl.BlockSpec((NC, BN, HEADS), lambda b: (0, b, 0)),
        ] + [_full(w.shape) for w in fw],
        out_specs=pl.BlockSpec((BN, D_OUT), lambda b: (b, 0)),
        out_shape=jax.ShapeDtypeStruct((N, D_OUT), f32),
    )(x, aggw, den_tab, *fw)
    return out
